# Initial kernel scaffold; baseline (speedup 1.0000x reference)
import jax
import jax.numpy as jnp
from jax.experimental import pallas as pl


def kernel(features, adj_indices, adj_values, adj2_indices, adj2_values, W_ft, b_ft, W1a, b1a, W1b, b1b, W2a, b2a, W2b, b2b, Wq, bq, Wk, bk, Wma, bma, Wmb, bmb):
    raise NotImplementedError("write your pallas kernel here")


# speedup vs baseline: 3.2341x; 3.2341x over previous
import functools
from functools import partial

import jax
import jax.numpy as jnp
from jax import lax
from jax.experimental import pallas as pl
from jax.experimental.pallas import tpu as pltpu
from jax.experimental.pallas import tpu_sc as plsc

N = 10000
E = 320000
D = 128
P = 64

NCORE = 2
NSUB = 16
BLK = 128
NB = 160
EPT = NB * BLK
EPAD = EPT * NSUB
CHUNK = 16
NPAD = 10240
ROWS_PER = NPAD // NSUB
RC = 2000


def _feat_matmul(x, w, b):
    def body(x_ref, w_ref, b_ref, o_ref):
        for i in range(N // RC):
            sl = pl.ds(i * RC, RC)
            o_ref[sl, :] = jnp.dot(x_ref[sl, :], w_ref[...]) + b_ref[...]

    return pl.pallas_call(
        body,
        out_shape=jax.ShapeDtypeStruct((N, D), jnp.float32),
    )(x, w, b.reshape(1, D))


def _spmm_sc(x, cols, rows, vals, zeros, zeros1):
    mesh = plsc.VectorSubcoreMesh(core_axis_name="c", subcore_axis_name="s")

    @functools.partial(
        pl.kernel,
        out_type=(jax.ShapeDtypeStruct((NCORE, NPAD, D), jnp.float32),
                  jax.ShapeDtypeStruct((NCORE, NPAD), jnp.float32)),
        mesh=mesh,
        scratch_types=[
            pltpu.VMEM((CHUNK, BLK), jnp.int32),
            pltpu.VMEM((CHUNK, BLK), jnp.int32),
            pltpu.VMEM((CHUNK, BLK), jnp.float32),
            pltpu.VMEM((BLK, D), jnp.float32),
            pltpu.VMEM_SHARED((NPAD, D), jnp.float32),
            pltpu.VMEM_SHARED((NPAD,), jnp.float32),
            pltpu.SemaphoreType.DMA,
        ],
    )
    def k(x_hbm, cols_hbm, rows_hbm, vals_hbm, zeros_hbm, zeros1_hbm,
          s_hbm, r_hbm, cols_v, rows_v, vals_v, gbuf, acc, acc1, sem):
        c = lax.axis_index("c")
        s = lax.axis_index("s")
        base = s * ROWS_PER
        pltpu.sync_copy(zeros_hbm.at[pl.ds(base, ROWS_PER)],
                        acc.at[pl.ds(base, ROWS_PER)])
        pltpu.sync_copy(zeros1_hbm.at[pl.ds(base, ROWS_PER)],
                        acc1.at[pl.ds(base, ROWS_PER)])
        plsc.subcore_barrier()

        def chunk_body(ch, carry):
            off = s * NB + ch * CHUNK
            pltpu.sync_copy(cols_hbm.at[c, pl.ds(off, CHUNK)], cols_v)
            pltpu.sync_copy(rows_hbm.at[c, pl.ds(off, CHUNK)], rows_v)
            pltpu.sync_copy(vals_hbm.at[c, pl.ds(off, CHUNK)], vals_v)

            def block(j, carry2):
                pltpu.async_copy(x_hbm.at[cols_v.at[j]], gbuf, sem).wait()

                def scale16(g, carry3):
                    vv = vals_v[j, pl.ds(g * 16, 16)]
                    for l in range(16):
                        v = vv[l]
                        e = g * 16 + l
                        for t in range(D // 16):
                            sl = pl.ds(t * 16, 16)
                            gbuf[e, sl] = gbuf[e, sl] * v
                    return carry3

                lax.fori_loop(0, BLK // 16, scale16, 0, unroll=False)
                pltpu.sync_copy(gbuf, acc.at[rows_v.at[j]], add=True)
                pltpu.sync_copy(vals_v.at[j], acc1.at[rows_v.at[j]], add=True)
                return carry2

            lax.fori_loop(0, CHUNK, block, 0, unroll=False)
            return carry

        lax.fori_loop(0, NB // CHUNK, chunk_body, 0, unroll=False)
        plsc.subcore_barrier()
        pltpu.sync_copy(acc.at[pl.ds(base, ROWS_PER)],
                        s_hbm.at[c, pl.ds(base, ROWS_PER)])
        pltpu.sync_copy(acc1.at[pl.ds(base, ROWS_PER)],
                        r_hbm.at[c, pl.ds(base, ROWS_PER)])

    return k(x, cols, rows, vals, zeros, zeros1)


def _combine(feat_in, a1_in, a2_in, W1a, b1a, W1b, b1b,
             W2a, b2a, W2b, b2b, Wq, bq, Wk, bk, Wma, bma, Wmb, bmb):

    def body(f_ref, a1_ref, a2_ref, w1a, b1a_,
             w1b, b1b_, w2a, b2a_, w2b, b2b_, wq, bq_, wk, bk_, wma, bma_,
             wmb, bmb_, o_ref):
        dot = jnp.dot
        for i in range(N // RC):
            sl = pl.ds(i * RC, RC)
            feat = f_ref[sl, :]
            a1 = a1_ref[sl, :]
            a2 = a2_ref[sl, :]
            h1 = jnp.maximum(dot(a1, w1a[...]) + b1a_[...], 0.0)
            h1 = dot(h1, w1b[...]) + b1b_[...]
            h2 = jnp.maximum(dot(a2, w2a[...]) + b2a_[...], 0.0)
            h2 = dot(h2, w2b[...]) + b2b_[...] + a2
            q = dot(feat, wq[...]) + bq_[...]
            k0 = dot(feat, wk[...]) + bk_[...]
            k1 = dot(h1, wk[...]) + bk_[...]
            k2 = dot(h2, wk[...]) + bk_[...]
            s0 = jnp.sum(q * k0, axis=1, keepdims=True)
            s1_ = jnp.sum(q * k1, axis=1, keepdims=True)
            s2_ = jnp.sum(q * k2, axis=1, keepdims=True)
            s3_ = jnp.sum(q * bk_[...], axis=1, keepdims=True)
            m = jnp.maximum(jnp.maximum(s0, s1_), jnp.maximum(s2_, s3_))
            e0 = jnp.exp(s0 - m)
            e1 = jnp.exp(s1_ - m)
            e2 = jnp.exp(s2_ - m)
            e3 = jnp.exp(s3_ - m)
            den = e0 + e1 + e2 + e3 + e3
            comb = (e0 * feat + e1 * h1 + e2 * h2) / den
            o = jnp.maximum(dot(comb, wma[...]) + bma_[...], 0.0)
            o_ref[sl, :] = dot(o, wmb[...]) + bmb_[...]

    return pl.pallas_call(
        body,
        out_shape=jax.ShapeDtypeStruct((N, D), jnp.float32),
    )(feat_in, a1_in, a2_in,
      W1a, b1a.reshape(1, D), W1b, b1b.reshape(1, D),
      W2a, b2a.reshape(1, D), W2b, b2b.reshape(1, D),
      Wq, bq.reshape(1, P), Wk, bk.reshape(1, P),
      Wma, bma.reshape(1, D), Wmb, bmb.reshape(1, D))


def kernel(features, adj_indices, adj_values, adj2_indices, adj2_values,
           W_ft, b_ft, W1a, b1a, W1b, b1b, W2a, b2a, W2b, b2b,
           Wq, bq, Wk, bk, Wma, bma, Wmb, bmb):
    pad_i = ((0, 0), (0, EPAD - E))
    cols = jnp.pad(jnp.stack([adj_indices[1], adj2_indices[1]]), pad_i)
    cols = cols.reshape(NCORE, NSUB * NB, BLK)
    rows = jnp.pad(jnp.stack([adj_indices[0], adj2_indices[0]]), pad_i)
    rows = rows.reshape(NCORE, NSUB * NB, BLK)
    vals = jnp.pad(jnp.stack([adj_values, adj2_values]), pad_i)
    vals = vals.reshape(NCORE, NSUB * NB, BLK)
    zeros = jnp.zeros((NPAD, D), jnp.float32)
    zeros1 = jnp.zeros((NPAD,), jnp.float32)
    feat = _feat_matmul(features, W_ft, b_ft)
    s_agg, _ = _spmm_sc(feat, cols, rows, vals, zeros, zeros1)
    return _combine(feat, s_agg[0, :N], s_agg[1, :N],
                    W1a, b1a, W1b, b1b, W2a, b2a, W2b, b2b,
                    Wq, bq, Wk, bk, Wma, bma, Wmb, bmb)


# speedup vs baseline: 9.1449x; 2.8276x over previous
import functools

import jax
import jax.numpy as jnp
from jax import lax
from jax.experimental import pallas as pl
from jax.experimental.pallas import tpu as pltpu
from jax.experimental.pallas import tpu_sc as plsc

N = 10000
E = 320000
D = 128
P = 64

NCORE = 2
NSUB = 16
BLK = 128
NB = 160
EPT = NB * BLK
EPAD = EPT * NSUB
CHUNK = 16
NPAD = 10240
ROWS_PER = NPAD // NSUB
RC = 2000


def _feat_matmul(x, w, b):
    def body(x_ref, w_ref, b_ref, o_ref):
        for i in range(N // RC):
            sl = pl.ds(i * RC, RC)
            o_ref[sl, :] = jnp.dot(x_ref[sl, :], w_ref[...]) + b_ref[...]

    return pl.pallas_call(
        body,
        out_shape=jax.ShapeDtypeStruct((N, D), jnp.float32),
    )(x, w, b.reshape(1, D))


def _spmm_sc(x, cols, rows, vals, zeros):
    mesh = plsc.VectorSubcoreMesh(core_axis_name="c", subcore_axis_name="s")

    @functools.partial(
        pl.kernel,
        out_type=jax.ShapeDtypeStruct((NCORE, NPAD, D), jnp.float32),
        mesh=mesh,
        scratch_types=[
            pltpu.VMEM((CHUNK, BLK), jnp.int32),
            pltpu.VMEM((CHUNK, BLK), jnp.int32),
            pltpu.VMEM((CHUNK, BLK), jnp.float32),
            pltpu.VMEM((BLK, D), jnp.float32),
            pltpu.VMEM((BLK, D), jnp.float32),
            pltpu.VMEM_SHARED((NPAD, D), jnp.float32),
            pltpu.SemaphoreType.DMA,
            pltpu.SemaphoreType.DMA,
        ],
    )
    def k(x_hbm, cols_hbm, rows_hbm, vals_hbm, zeros_hbm,
          s_hbm, cols_v, rows_v, vals_v, g0, g1, acc, sem0, sem1):
        c = lax.axis_index("c")
        s = lax.axis_index("s")
        base = s * ROWS_PER
        gbufs = (g0, g1)
        sems = (sem0, sem1)
        pltpu.sync_copy(zeros_hbm.at[pl.ds(base, ROWS_PER)],
                        acc.at[pl.ds(base, ROWS_PER)])
        plsc.subcore_barrier()

        def scale16(gb, j):
            def body(g, carry):
                vv = vals_v[j, pl.ds(g * 16, 16)]
                for l in range(16):
                    v = vv[l]
                    e = g * 16 + l
                    for t in range(D // 16):
                        sl = pl.ds(t * 16, 16)
                        gb[e, sl] = gb[e, sl] * v
                return carry

            lax.fori_loop(0, BLK // 16, body, 0, unroll=False)

        def chunk_body(ch, carry):
            off = s * NB + ch * CHUNK
            pltpu.sync_copy(cols_hbm.at[c, pl.ds(off, CHUNK)], cols_v)
            pltpu.sync_copy(rows_hbm.at[c, pl.ds(off, CHUNK)], rows_v)
            pltpu.sync_copy(vals_hbm.at[c, pl.ds(off, CHUNK)], vals_v)
            pltpu.async_copy(x_hbm.at[cols_v.at[0]], g0, sem0)

            def pair(jp, carry2):
                for b in range(2):
                    j = jp * 2 + b
                    pltpu.make_async_copy(
                        x_hbm.at[cols_v.at[j]], gbufs[b], sems[b]).wait()
                    nj = j + 1

                    @pl.when(nj < CHUNK)
                    def _prefetch():
                        pltpu.async_copy(
                            x_hbm.at[cols_v.at[nj]], gbufs[1 - b],
                            sems[1 - b])

                    scale16(gbufs[b], j)
                    pltpu.sync_copy(gbufs[b], acc.at[rows_v.at[j]], add=True)
                return carry2

            lax.fori_loop(0, CHUNK // 2, pair, 0, unroll=False)
            return carry

        lax.fori_loop(0, NB // CHUNK, chunk_body, 0, unroll=False)
        plsc.subcore_barrier()
        pltpu.sync_copy(acc.at[pl.ds(base, ROWS_PER)],
                        s_hbm.at[c, pl.ds(base, ROWS_PER)])

    return k(x, cols, rows, vals, zeros)


def _combine(feat_in, a1_in, a2_in, W1a, b1a, W1b, b1b,
             W2a, b2a, W2b, b2b, Wq, bq, Wk, bk, Wma, bma, Wmb, bmb):

    def body(f_ref, a1_ref, a2_ref, w1a, b1a_,
             w1b, b1b_, w2a, b2a_, w2b, b2b_, wq, bq_, wk, bk_, wma, bma_,
             wmb, bmb_, o_ref):
        dot = jnp.dot
        for i in range(N // RC):
            sl = pl.ds(i * RC, RC)
            feat = f_ref[sl, :]
            a1 = a1_ref[sl, :]
            a2 = a2_ref[sl, :]
            h1 = jnp.maximum(dot(a1, w1a[...]) + b1a_[...], 0.0)
            h1 = dot(h1, w1b[...]) + b1b_[...]
            h2 = jnp.maximum(dot(a2, w2a[...]) + b2a_[...], 0.0)
            h2 = dot(h2, w2b[...]) + b2b_[...] + a2
            q = dot(feat, wq[...]) + bq_[...]
            k0 = dot(feat, wk[...]) + bk_[...]
            k1 = dot(h1, wk[...]) + bk_[...]
            k2 = dot(h2, wk[...]) + bk_[...]
            s0 = jnp.sum(q * k0, axis=1, keepdims=True)
            s1_ = jnp.sum(q * k1, axis=1, keepdims=True)
            s2_ = jnp.sum(q * k2, axis=1, keepdims=True)
            s3_ = jnp.sum(q * bk_[...], axis=1, keepdims=True)
            m = jnp.maximum(jnp.maximum(s0, s1_), jnp.maximum(s2_, s3_))
            e0 = jnp.exp(s0 - m)
            e1 = jnp.exp(s1_ - m)
            e2 = jnp.exp(s2_ - m)
            e3 = jnp.exp(s3_ - m)
            den = e0 + e1 + e2 + e3 + e3
            comb = (e0 * feat + e1 * h1 + e2 * h2) / den
            o = jnp.maximum(dot(comb, wma[...]) + bma_[...], 0.0)
            o_ref[sl, :] = dot(o, wmb[...]) + bmb_[...]

    return pl.pallas_call(
        body,
        out_shape=jax.ShapeDtypeStruct((N, D), jnp.float32),
    )(feat_in, a1_in, a2_in,
      W1a, b1a.reshape(1, D), W1b, b1b.reshape(1, D),
      W2a, b2a.reshape(1, D), W2b, b2b.reshape(1, D),
      Wq, bq.reshape(1, P), Wk, bk.reshape(1, P),
      Wma, bma.reshape(1, D), Wmb, bmb.reshape(1, D))


def kernel(features, adj_indices, adj_values, adj2_indices, adj2_values,
           W_ft, b_ft, W1a, b1a, W1b, b1b, W2a, b2a, W2b, b2b,
           Wq, bq, Wk, bk, Wma, bma, Wmb, bmb):
    spread = jnp.tile(jnp.arange(EPAD - E, dtype=jnp.int32) % N, (2, 1))
    cols = jnp.concatenate(
        [jnp.stack([adj_indices[1], adj2_indices[1]]), spread], axis=1)
    cols = cols.reshape(NCORE, NSUB * NB, BLK)
    rows = jnp.concatenate(
        [jnp.stack([adj_indices[0], adj2_indices[0]]), spread], axis=1)
    rows = rows.reshape(NCORE, NSUB * NB, BLK)
    vals = jnp.pad(jnp.stack([adj_values, adj2_values]),
                   ((0, 0), (0, EPAD - E)))
    vals = vals.reshape(NCORE, NSUB * NB, BLK)
    zeros = jnp.zeros((NPAD, D), jnp.float32)
    feat = _feat_matmul(features, W_ft, b_ft)
    s_agg = _spmm_sc(feat, cols, rows, vals, zeros)
    return _combine(feat, s_agg[0, :N], s_agg[1, :N],
                    W1a, b1a, W1b, b1b, W2a, b2a, W2b, b2b,
                    Wq, bq, Wk, bk, Wma, bma, Wmb, bmb)


# speedup vs baseline: 9.4723x; 1.0358x over previous
import functools

import jax
import jax.numpy as jnp
from jax import lax
from jax.experimental import pallas as pl
from jax.experimental.pallas import tpu as pltpu
from jax.experimental.pallas import tpu_sc as plsc

N = 10000
E = 320000
D = 128
P = 64

NCORE = 2
NSUB = 16
BLK = 128
NB = 160
EPT = NB * BLK
EPAD = EPT * NSUB
CHUNK = 32
NPAD = 10240
ROWS_PER = NPAD // NSUB
RC = 2000


def _feat_matmul(x, w, b):
    def body(x_ref, w_ref, b_ref, o_ref):
        for i in range(N // RC):
            sl = pl.ds(i * RC, RC)
            o_ref[sl, :] = jnp.dot(x_ref[sl, :], w_ref[...]) + b_ref[...]

    return pl.pallas_call(
        body,
        out_shape=jax.ShapeDtypeStruct((N, D), jnp.float32),
    )(x, w, b.reshape(1, D))


def _spmm_sc(x, cols, rows, vals, zeros):
    mesh = plsc.VectorSubcoreMesh(core_axis_name="c", subcore_axis_name="s")

    @functools.partial(
        pl.kernel,
        out_type=jax.ShapeDtypeStruct((NCORE, NPAD, D), jnp.float32),
        mesh=mesh,
        scratch_types=[
            pltpu.VMEM((CHUNK, BLK), jnp.int32),
            pltpu.VMEM((CHUNK, BLK), jnp.int32),
            pltpu.VMEM((CHUNK, BLK), jnp.float32),
            pltpu.VMEM((BLK, D), jnp.float32),
            pltpu.VMEM((BLK, D), jnp.float32),
            pltpu.VMEM_SHARED((NPAD, D), jnp.float32),
            pltpu.SemaphoreType.DMA,
            pltpu.SemaphoreType.DMA,
        ],
    )
    def k(x_hbm, cols_hbm, rows_hbm, vals_hbm, zeros_hbm,
          s_hbm, cols_v, rows_v, vals_v, g0, g1, acc, sem0, sem1):
        c = lax.axis_index("c")
        s = lax.axis_index("s")
        base = s * ROWS_PER
        gbufs = (g0, g1)
        sems = (sem0, sem1)
        pltpu.sync_copy(zeros_hbm.at[pl.ds(base, ROWS_PER)],
                        acc.at[pl.ds(base, ROWS_PER)])
        plsc.subcore_barrier()

        def scale16(gb, j):
            def body(g, carry):
                vv = vals_v[j, pl.ds(g * 16, 16)]
                for l in range(16):
                    v = vv[l]
                    e = g * 16 + l
                    for t in range(D // 16):
                        sl = pl.ds(t * 16, 16)
                        gb[e, sl] = gb[e, sl] * v
                return carry

            lax.fori_loop(0, BLK // 16, body, 0, unroll=False)

        def chunk_body(ch, carry):
            off = s * NB + ch * CHUNK
            pltpu.sync_copy(cols_hbm.at[c, pl.ds(off, CHUNK)], cols_v)
            pltpu.sync_copy(rows_hbm.at[c, pl.ds(off, CHUNK)], rows_v)
            pltpu.sync_copy(vals_hbm.at[c, pl.ds(off, CHUNK)], vals_v)
            pltpu.async_copy(x_hbm.at[cols_v.at[0]], g0, sem0)

            def pair(jp, carry2):
                for b in range(2):
                    j = jp * 2 + b
                    pltpu.make_async_copy(
                        x_hbm.at[cols_v.at[j]], gbufs[b], sems[b]).wait()
                    nj = j + 1

                    @pl.when(nj < CHUNK)
                    def _prefetch():
                        pltpu.async_copy(
                            x_hbm.at[cols_v.at[nj]], gbufs[1 - b],
                            sems[1 - b])

                    scale16(gbufs[b], j)
                    pltpu.sync_copy(gbufs[b], acc.at[rows_v.at[j]], add=True)
                return carry2

            lax.fori_loop(0, CHUNK // 2, pair, 0, unroll=False)
            return carry

        lax.fori_loop(0, NB // CHUNK, chunk_body, 0, unroll=False)
        plsc.subcore_barrier()
        pltpu.sync_copy(acc.at[pl.ds(base, ROWS_PER)],
                        s_hbm.at[c, pl.ds(base, ROWS_PER)])

    return k(x, cols, rows, vals, zeros)


def _combine(feat_in, a1_in, a2_in, W1a, b1a, W1b, b1b,
             W2a, b2a, W2b, b2b, Wq, bq, Wk, bk, Wma, bma, Wmb, bmb):

    def body(f_ref, a1_ref, a2_ref, w1a, b1a_,
             w1b, b1b_, w2a, b2a_, w2b, b2b_, wq, bq_, wk, bk_, wma, bma_,
             wmb, bmb_, o_ref):
        dot = jnp.dot
        for i in range(N // RC):
            sl = pl.ds(i * RC, RC)
            feat = f_ref[sl, :]
            a1 = a1_ref[sl, :]
            a2 = a2_ref[sl, :]
            h1 = jnp.maximum(dot(a1, w1a[...]) + b1a_[...], 0.0)
            h1 = dot(h1, w1b[...]) + b1b_[...]
            h2 = jnp.maximum(dot(a2, w2a[...]) + b2a_[...], 0.0)
            h2 = dot(h2, w2b[...]) + b2b_[...] + a2
            q = dot(feat, wq[...]) + bq_[...]
            k0 = dot(feat, wk[...]) + bk_[...]
            k1 = dot(h1, wk[...]) + bk_[...]
            k2 = dot(h2, wk[...]) + bk_[...]
            s0 = jnp.sum(q * k0, axis=1, keepdims=True)
            s1_ = jnp.sum(q * k1, axis=1, keepdims=True)
            s2_ = jnp.sum(q * k2, axis=1, keepdims=True)
            s3_ = jnp.sum(q * bk_[...], axis=1, keepdims=True)
            m = jnp.maximum(jnp.maximum(s0, s1_), jnp.maximum(s2_, s3_))
            e0 = jnp.exp(s0 - m)
            e1 = jnp.exp(s1_ - m)
            e2 = jnp.exp(s2_ - m)
            e3 = jnp.exp(s3_ - m)
            den = e0 + e1 + e2 + e3 + e3
            comb = (e0 * feat + e1 * h1 + e2 * h2) / den
            o = jnp.maximum(dot(comb, wma[...]) + bma_[...], 0.0)
            o_ref[sl, :] = dot(o, wmb[...]) + bmb_[...]

    return pl.pallas_call(
        body,
        out_shape=jax.ShapeDtypeStruct((N, D), jnp.float32),
    )(feat_in, a1_in, a2_in,
      W1a, b1a.reshape(1, D), W1b, b1b.reshape(1, D),
      W2a, b2a.reshape(1, D), W2b, b2b.reshape(1, D),
      Wq, bq.reshape(1, P), Wk, bk.reshape(1, P),
      Wma, bma.reshape(1, D), Wmb, bmb.reshape(1, D))


def kernel(features, adj_indices, adj_values, adj2_indices, adj2_values,
           W_ft, b_ft, W1a, b1a, W1b, b1b, W2a, b2a, W2b, b2b,
           Wq, bq, Wk, bk, Wma, bma, Wmb, bmb):
    spread = jnp.tile(jnp.arange(EPAD - E, dtype=jnp.int32) % N, (2, 1))
    cols = jnp.concatenate(
        [jnp.stack([adj_indices[1], adj2_indices[1]]), spread], axis=1)
    cols = cols.reshape(NCORE, NSUB * NB, BLK)
    rows = jnp.concatenate(
        [jnp.stack([adj_indices[0], adj2_indices[0]]), spread], axis=1)
    rows = rows.reshape(NCORE, NSUB * NB, BLK)
    vals = jnp.pad(jnp.stack([adj_values, adj2_values]),
                   ((0, 0), (0, EPAD - E)))
    vals = vals.reshape(NCORE, NSUB * NB, BLK)
    zeros = jnp.zeros((NPAD, D), jnp.float32)
    feat = _feat_matmul(features, W_ft, b_ft)
    s_agg = _spmm_sc(feat, cols, rows, vals, zeros)
    return _combine(feat, s_agg[0, :N], s_agg[1, :N],
                    W1a, b1a, W1b, b1b, W2a, b2a, W2b, b2b,
                    Wq, bq, Wk, bk, Wma, bma, Wmb, bmb)


# speedup vs baseline: 9.5463x; 1.0078x over previous
import functools

import jax
import jax.numpy as jnp
from jax import lax
from jax.experimental import pallas as pl
from jax.experimental.pallas import tpu as pltpu
from jax.experimental.pallas import tpu_sc as plsc

N = 10000
E = 320000
D = 128
P = 64

NCORE = 2
NSUB = 16
BLK = 128
NB = 160
EPT = NB * BLK
EPAD = EPT * NSUB
CHUNK = 40
NPAD = 10240
ROWS_PER = NPAD // NSUB
RC = 2000


def _feat_matmul(x, w, b):
    def body(x_ref, w_ref, b_ref, o_ref):
        for i in range(N // RC):
            sl = pl.ds(i * RC, RC)
            o_ref[sl, :] = jnp.dot(x_ref[sl, :], w_ref[...]) + b_ref[...]

    return pl.pallas_call(
        body,
        out_shape=jax.ShapeDtypeStruct((N, D), jnp.float32),
    )(x, w, b.reshape(1, D))


def _spmm_sc(x, cols, rows, vals, zeros):
    mesh = plsc.VectorSubcoreMesh(core_axis_name="c", subcore_axis_name="s")

    @functools.partial(
        pl.kernel,
        out_type=jax.ShapeDtypeStruct((NCORE, NPAD, D), jnp.float32),
        mesh=mesh,
        scratch_types=[
            pltpu.VMEM((CHUNK, BLK), jnp.int32),
            pltpu.VMEM((CHUNK, BLK), jnp.int32),
            pltpu.VMEM((CHUNK, BLK), jnp.float32),
            pltpu.VMEM((BLK, D), jnp.float32),
            pltpu.VMEM((BLK, D), jnp.float32),
            pltpu.VMEM_SHARED((NPAD, D), jnp.float32),
            pltpu.SemaphoreType.DMA,
            pltpu.SemaphoreType.DMA,
        ],
    )
    def k(x_hbm, cols_hbm, rows_hbm, vals_hbm, zeros_hbm,
          s_hbm, cols_v, rows_v, vals_v, g0, g1, acc, sem0, sem1):
        c = lax.axis_index("c")
        s = lax.axis_index("s")
        base = s * ROWS_PER
        gbufs = (g0, g1)
        sems = (sem0, sem1)
        pltpu.sync_copy(zeros_hbm.at[pl.ds(base, ROWS_PER)],
                        acc.at[pl.ds(base, ROWS_PER)])
        plsc.subcore_barrier()

        def scale16(gb, j):
            def body(g, carry):
                vv = vals_v[j, pl.ds(g * 16, 16)]
                for l in range(16):
                    v = vv[l]
                    e = g * 16 + l
                    for t in range(D // 16):
                        sl = pl.ds(t * 16, 16)
                        gb[e, sl] = gb[e, sl] * v
                return carry

            lax.fori_loop(0, BLK // 16, body, 0, unroll=False)

        def chunk_body(ch, carry):
            off = s * NB + ch * CHUNK
            pltpu.sync_copy(cols_hbm.at[c, pl.ds(off, CHUNK)], cols_v)
            pltpu.sync_copy(rows_hbm.at[c, pl.ds(off, CHUNK)], rows_v)
            pltpu.sync_copy(vals_hbm.at[c, pl.ds(off, CHUNK)], vals_v)
            pltpu.async_copy(x_hbm.at[cols_v.at[0]], g0, sem0)

            def pair(jp, carry2):
                for b in range(2):
                    j = jp * 2 + b
                    pltpu.make_async_copy(
                        x_hbm.at[cols_v.at[j]], gbufs[b], sems[b]).wait()
                    nj = j + 1

                    @pl.when(nj < CHUNK)
                    def _prefetch():
                        pltpu.async_copy(
                            x_hbm.at[cols_v.at[nj]], gbufs[1 - b],
                            sems[1 - b])

                    scale16(gbufs[b], j)
                    pltpu.sync_copy(gbufs[b], acc.at[rows_v.at[j]], add=True)
                return carry2

            lax.fori_loop(0, CHUNK // 2, pair, 0, unroll=False)
            return carry

        lax.fori_loop(0, NB // CHUNK, chunk_body, 0, unroll=False)
        plsc.subcore_barrier()
        pltpu.sync_copy(acc.at[pl.ds(base, ROWS_PER)],
                        s_hbm.at[c, pl.ds(base, ROWS_PER)])

    return k(x, cols, rows, vals, zeros)


def _combine(feat_in, a1_in, a2_in, W1a, b1a, W1b, b1b,
             W2a, b2a, W2b, b2b, Wq, bq, Wk, bk, Wma, bma, Wmb, bmb):

    def body(f_ref, a1_ref, a2_ref, w1a, b1a_,
             w1b, b1b_, w2a, b2a_, w2b, b2b_, wq, bq_, wk, bk_, wma, bma_,
             wmb, bmb_, o_ref):
        dot = jnp.dot
        for i in range(N // RC):
            sl = pl.ds(i * RC, RC)
            feat = f_ref[sl, :]
            a1 = a1_ref[sl, :]
            a2 = a2_ref[sl, :]
            h1 = jnp.maximum(dot(a1, w1a[...]) + b1a_[...], 0.0)
            h1 = dot(h1, w1b[...]) + b1b_[...]
            h2 = jnp.maximum(dot(a2, w2a[...]) + b2a_[...], 0.0)
            h2 = dot(h2, w2b[...]) + b2b_[...] + a2
            q = dot(feat, wq[...]) + bq_[...]
            k0 = dot(feat, wk[...]) + bk_[...]
            k1 = dot(h1, wk[...]) + bk_[...]
            k2 = dot(h2, wk[...]) + bk_[...]
            s0 = jnp.sum(q * k0, axis=1, keepdims=True)
            s1_ = jnp.sum(q * k1, axis=1, keepdims=True)
            s2_ = jnp.sum(q * k2, axis=1, keepdims=True)
            s3_ = jnp.sum(q * bk_[...], axis=1, keepdims=True)
            m = jnp.maximum(jnp.maximum(s0, s1_), jnp.maximum(s2_, s3_))
            e0 = jnp.exp(s0 - m)
            e1 = jnp.exp(s1_ - m)
            e2 = jnp.exp(s2_ - m)
            e3 = jnp.exp(s3_ - m)
            den = e0 + e1 + e2 + e3 + e3
            comb = (e0 * feat + e1 * h1 + e2 * h2) / den
            o = jnp.maximum(dot(comb, wma[...]) + bma_[...], 0.0)
            o_ref[sl, :] = dot(o, wmb[...]) + bmb_[...]

    return pl.pallas_call(
        body,
        out_shape=jax.ShapeDtypeStruct((N, D), jnp.float32),
    )(feat_in, a1_in, a2_in,
      W1a, b1a.reshape(1, D), W1b, b1b.reshape(1, D),
      W2a, b2a.reshape(1, D), W2b, b2b.reshape(1, D),
      Wq, bq.reshape(1, P), Wk, bk.reshape(1, P),
      Wma, bma.reshape(1, D), Wmb, bmb.reshape(1, D))


def kernel(features, adj_indices, adj_values, adj2_indices, adj2_values,
           W_ft, b_ft, W1a, b1a, W1b, b1b, W2a, b2a, W2b, b2b,
           Wq, bq, Wk, bk, Wma, bma, Wmb, bmb):
    spread = jnp.tile(jnp.arange(EPAD - E, dtype=jnp.int32) % N, (2, 1))
    cols = jnp.concatenate(
        [jnp.stack([adj_indices[1], adj2_indices[1]]), spread], axis=1)
    cols = cols.reshape(NCORE, NSUB * NB, BLK)
    rows = jnp.concatenate(
        [jnp.stack([adj_indices[0], adj2_indices[0]]), spread], axis=1)
    rows = rows.reshape(NCORE, NSUB * NB, BLK)
    vals = jnp.pad(jnp.stack([adj_values, adj2_values]),
                   ((0, 0), (0, EPAD - E)))
    vals = vals.reshape(NCORE, NSUB * NB, BLK)
    zeros = jnp.zeros((NPAD, D), jnp.float32)
    feat = _feat_matmul(features, W_ft, b_ft)
    s_agg = _spmm_sc(feat, cols, rows, vals, zeros)
    return _combine(feat, s_agg[0, :N], s_agg[1, :N],
                    W1a, b1a, W1b, b1b, W2a, b2a, W2b, b2b,
                    Wq, bq, Wk, bk, Wma, bma, Wmb, bmb)
